# baseline (speedup 1.0000x reference)
import jax
import jax.numpy as jnp
from jax import lax
from jax.experimental import pallas as pl
from jax.experimental.pallas import tpu as pltpu
from jax.experimental.pallas import tpu_sc as plsc

N_CORES = 2
N_SUBCORES = 16
NW = N_CORES * N_SUBCORES
L = 16
N_CHUNK = 4
N_BUF = 2


def _planar_body(m_hbm, s_hbm, wu_hbm, b_hbm, out_hbm, *scratch):
    idx_b = scratch[0:N_BUF]
    wu_b = scratch[N_BUF:2 * N_BUF]
    bb_b = scratch[2 * N_BUF:3 * N_BUF]
    s_b = scratch[3 * N_BUF:4 * N_BUF]
    sem_b = scratch[4 * N_BUF:5 * N_BUF]
    osem_b = scratch[5 * N_BUF:6 * N_BUF]

    bpc = idx_b[0].shape[0]
    sdim = s_b[0].shape[1]
    nj = sdim // L
    wid = lax.axis_index("s") * N_CORES + lax.axis_index("c")
    base0 = wid * (N_CHUNK * bpc)

    out_cp = [None] * N_BUF

    def chunk_start(c):
        i = c % N_BUF
        if out_cp[i] is not None:
            out_cp[i].wait()
            out_cp[i] = None
        base = base0 + c * bpc
        pltpu.sync_copy(m_hbm.at[pl.ds(base, bpc)], idx_b[i])
        cps = (pltpu.async_copy(wu_hbm.at[idx_b[i]], wu_b[i], sem_b[i]),
               pltpu.async_copy(b_hbm.at[idx_b[i]],
                                bb_b[i].at[pl.ds(0, bpc)], sem_b[i]),
               pltpu.async_copy(s_hbm.at[pl.ds(base, bpc)], s_b[i], sem_b[i]))
        return cps

    def chunk_finish(c, cps):
        i = c % N_BUF
        for cp in cps:
            cp.wait()
        wu_v, b_v, s_v = wu_b[i], bb_b[i], s_b[i]
        lanes = lax.iota(jnp.int32, L)

        def group(g, carry):
            uw_p = jnp.zeros((L,), jnp.float32)
            ww_p = jnp.zeros((L,), jnp.float32)
            sw_p = jnp.zeros((L,), jnp.float32)
            r0 = g * L
            for k in range(L):
                r = r0 + k
                uw = jnp.zeros((L,), jnp.float32)
                ww = jnp.zeros((L,), jnp.float32)
                sw = jnp.zeros((L,), jnp.float32)
                for j in range(nj):
                    wj = wu_v[r, pl.ds(j * L, L)]
                    uw = uw + wu_v[r, pl.ds(sdim + j * L, L)] * wj
                    ww = ww + wj * wj
                    sw = sw + s_v[r, pl.ds(j * L, L)] * wj
                mk = lanes == k
                uw_p = jnp.where(mk, jnp.full((L,), jnp.sum(uw)), uw_p)
                ww_p = jnp.where(mk, jnp.full((L,), jnp.sum(ww)), ww_p)
                sw_p = jnp.where(mk, jnp.full((L,), jnp.sum(sw)), sw_p)
            x_p = sw_p + b_v[pl.ds(r0, L)]

            z = jnp.exp(jnp.minimum(uw_p, -1.0))
            y0 = z * (1.0 - z * (0.5 - z * (1.0 / 3.0 - z * 0.25)))
            y1 = y0 - 1.0 + (1.0 + z) * jnp.exp(-y0)
            scale_p = jnp.where(uw_p < -1.0, (y1 - 1.0 - uw_p) / ww_p, 0.0)

            e = jnp.exp(-2.0 * jnp.abs(x_p))
            t0 = (1.0 - e) / (1.0 + e)
            t_p = jnp.where(x_p < 0.0, -t0, t0)

            for k in range(L):
                r = r0 + k
                mk = lanes == k
                scale = jnp.full((L,), jnp.sum(jnp.where(mk, scale_p, 0.0)))
                t = jnp.full((L,), jnp.sum(jnp.where(mk, t_p, 0.0)))
                for j in range(nj):
                    sl = pl.ds(j * L, L)
                    s_v[r, sl] = s_v[r, sl] + (wu_v[r, pl.ds(sdim + j * L, L)]
                                               + scale * wu_v[r, sl]) * t
            return carry

        lax.fori_loop(0, bpc // L, group, 0)

        out_cp[i] = pltpu.async_copy(
            s_v, out_hbm.at[pl.ds(base0 + c * bpc, bpc)], osem_b[i])

    cps = chunk_start(0)
    for c in range(N_CHUNK):
        nxt = chunk_start(c + 1) if c + 1 < N_CHUNK else None
        chunk_finish(c, cps)
        cps = nxt
    for i in range(N_BUF):
        if out_cp[i] is not None:
            out_cp[i].wait()


def kernel(m, s, w, b, u):
    batch, sdim = s.shape
    n_rows = w.shape[0]
    bpc = batch // (NW * N_CHUNK)
    wu_tab = jnp.concatenate([w.T, u.T], axis=0).T
    b_flat = b.reshape(n_rows)

    scratch = (
        [pltpu.VMEM((bpc,), jnp.int32) for _ in range(N_BUF)] +
        [pltpu.VMEM((bpc, 2 * sdim), jnp.float32) for _ in range(N_BUF)] +
        [pltpu.VMEM((bpc + L,), jnp.float32) for _ in range(N_BUF)] +
        [pltpu.VMEM((bpc, sdim), jnp.float32) for _ in range(N_BUF)] +
        [pltpu.SemaphoreType.DMA for _ in range(N_BUF)] +
        [pltpu.SemaphoreType.DMA for _ in range(N_BUF)]
    )
    run = pl.kernel(
        _planar_body,
        out_type=jax.ShapeDtypeStruct((batch, sdim), jnp.float32),
        mesh=plsc.VectorSubcoreMesh(core_axis_name="c", subcore_axis_name="s"),
        compiler_params=pltpu.CompilerParams(
            needs_layout_passes=False, use_tc_tiling_on_sc=True),
        scratch_types=scratch,
    )
    return run(m, s, wu_tab, b_flat)


# speedup vs baseline: 1.0074x; 1.0074x over previous
import jax
import jax.numpy as jnp
from jax import lax
from jax.experimental import pallas as pl
from jax.experimental.pallas import tpu as pltpu
from jax.experimental.pallas import tpu_sc as plsc

N_CORES = 2
N_SUBCORES = 16
NW = N_CORES * N_SUBCORES
L = 16
N_CHUNK = 4
N_BUF = 2


def _planar_body(m_hbm, s_hbm, wu_hbm, b_hbm, out_hbm, *scratch):
    idx_b = scratch[0:N_BUF]
    wu_b = scratch[N_BUF:2 * N_BUF]
    bb_b = scratch[2 * N_BUF:3 * N_BUF]
    s_b = scratch[3 * N_BUF:4 * N_BUF]
    sem_b = scratch[4 * N_BUF:5 * N_BUF]
    osem_b = scratch[5 * N_BUF:6 * N_BUF]

    bpc = idx_b[0].shape[0]
    sdim = s_b[0].shape[1]
    nj = sdim // L
    wid = lax.axis_index("s") * N_CORES + lax.axis_index("c")
    base0 = wid * (N_CHUNK * bpc)

    out_cp = [None] * N_BUF

    def chunk_start(c):
        i = c % N_BUF
        if out_cp[i] is not None:
            out_cp[i].wait()
            out_cp[i] = None
        base = base0 + c * bpc
        pltpu.sync_copy(m_hbm.at[pl.ds(base, bpc)], idx_b[i])
        cps = (pltpu.async_copy(wu_hbm.at[idx_b[i]], wu_b[i], sem_b[i]),
               pltpu.async_copy(b_hbm.at[idx_b[i]],
                                bb_b[i].at[pl.ds(0, bpc)], sem_b[i]),
               pltpu.async_copy(s_hbm.at[pl.ds(base, bpc)], s_b[i], sem_b[i]))
        return cps

    def chunk_finish(c, cps):
        i = c % N_BUF
        for cp in cps:
            cp.wait()
        wu_v, b_v, s_v = wu_b[i], bb_b[i], s_b[i]
        lanes = lax.iota(jnp.int32, L)

        def group(g, carry):
            uw_p = jnp.zeros((L,), jnp.float32)
            ww_p = jnp.zeros((L,), jnp.float32)
            sw_p = jnp.zeros((L,), jnp.float32)
            r0 = g * L
            for k in range(L):
                r = r0 + k
                uw = jnp.zeros((L,), jnp.float32)
                ww = jnp.zeros((L,), jnp.float32)
                sw = jnp.zeros((L,), jnp.float32)
                for j in range(nj):
                    wj = wu_v[r, pl.ds(j * L, L)]
                    uw = uw + wu_v[r, pl.ds(sdim + j * L, L)] * wj
                    ww = ww + wj * wj
                    sw = sw + s_v[r, pl.ds(j * L, L)] * wj
                mk = lanes == k
                uw_p = jnp.where(mk, jnp.full((L,), jnp.sum(uw)), uw_p)
                ww_p = jnp.where(mk, jnp.full((L,), jnp.sum(ww)), ww_p)
                sw_p = jnp.where(mk, jnp.full((L,), jnp.sum(sw)), sw_p)
            x_p = sw_p + b_v[pl.ds(r0, L)]

            z = jnp.exp(jnp.minimum(uw_p, -1.0))
            y0 = z * (1.0 - z * (0.5 - z * (1.0 / 3.0 - z * 0.25)))
            y1 = y0 - 1.0 + (1.0 + z) * jnp.exp(-y0)
            scale_p = jnp.where(uw_p < -1.0, (y1 - 1.0 - uw_p) / ww_p, 0.0)

            e = jnp.exp(-2.0 * jnp.abs(x_p))
            t0 = (1.0 - e) / (1.0 + e)
            t_p = jnp.where(x_p < 0.0, -t0, t0)

            for k in range(L):
                r = r0 + k
                mk = lanes == k
                scale = jnp.full((L,), jnp.sum(jnp.where(mk, scale_p, 0.0)))
                t = jnp.full((L,), jnp.sum(jnp.where(mk, t_p, 0.0)))
                for j in range(nj):
                    sl = pl.ds(j * L, L)
                    s_v[r, sl] = s_v[r, sl] + (wu_v[r, pl.ds(sdim + j * L, L)]
                                               + scale * wu_v[r, sl]) * t
            return carry

        lax.fori_loop(0, bpc // L, group, 0)

        out_cp[i] = pltpu.async_copy(
            s_v, out_hbm.at[pl.ds(base0 + c * bpc, bpc)], osem_b[i])

    cps = chunk_start(0)
    for c in range(N_CHUNK):
        nxt = chunk_start(c + 1) if c + 1 < N_CHUNK else None
        chunk_finish(c, cps)
        cps = nxt
    for i in range(N_BUF):
        if out_cp[i] is not None:
            out_cp[i].wait()


def kernel(m, s, w, b, u):
    batch, sdim = s.shape
    n_rows = w.shape[0]
    bpc = batch // (NW * N_CHUNK)
    wu_tab = jnp.concatenate([w, u], axis=1)
    b_flat = b.reshape(n_rows)

    scratch = (
        [pltpu.VMEM((bpc,), jnp.int32) for _ in range(N_BUF)] +
        [pltpu.VMEM((bpc, 2 * sdim), jnp.float32) for _ in range(N_BUF)] +
        [pltpu.VMEM((bpc + L,), jnp.float32) for _ in range(N_BUF)] +
        [pltpu.VMEM((bpc, sdim), jnp.float32) for _ in range(N_BUF)] +
        [pltpu.SemaphoreType.DMA for _ in range(N_BUF)] +
        [pltpu.SemaphoreType.DMA for _ in range(N_BUF)]
    )
    run = pl.kernel(
        _planar_body,
        out_type=jax.ShapeDtypeStruct((batch, sdim), jnp.float32),
        mesh=plsc.VectorSubcoreMesh(core_axis_name="c", subcore_axis_name="s"),
        compiler_params=pltpu.CompilerParams(
            needs_layout_passes=False, use_tc_tiling_on_sc=True),
        scratch_types=scratch,
    )
    return run(m, s, wu_tab, b_flat)


# speedup vs baseline: 1.0159x; 1.0084x over previous
import jax
import jax.numpy as jnp
from jax import lax
from jax.experimental import pallas as pl
from jax.experimental.pallas import tpu as pltpu
from jax.experimental.pallas import tpu_sc as plsc

N_CORES = 2
N_SUBCORES = 16
NW = N_CORES * N_SUBCORES
L = 16
N_CHUNK = 4
N_BUF = 3


def _planar_body(m_hbm, s_hbm, wu_hbm, b_hbm, out_hbm, *scratch):
    idx_b = scratch[0:N_BUF]
    wu_b = scratch[N_BUF:2 * N_BUF]
    bb_b = scratch[2 * N_BUF:3 * N_BUF]
    s_b = scratch[3 * N_BUF:4 * N_BUF]
    sem_b = scratch[4 * N_BUF:5 * N_BUF]
    osem_b = scratch[5 * N_BUF:6 * N_BUF]

    bpc = idx_b[0].shape[0]
    sdim = s_b[0].shape[1]
    nj = sdim // L
    wid = lax.axis_index("s") * N_CORES + lax.axis_index("c")
    base0 = wid * (N_CHUNK * bpc)

    out_cp = [None] * N_BUF

    def chunk_start(c):
        i = c % N_BUF
        if out_cp[i] is not None:
            out_cp[i].wait()
            out_cp[i] = None
        base = base0 + c * bpc
        pltpu.sync_copy(m_hbm.at[pl.ds(base, bpc)], idx_b[i])
        cps = (pltpu.async_copy(wu_hbm.at[idx_b[i]], wu_b[i], sem_b[i]),
               pltpu.async_copy(b_hbm.at[idx_b[i]],
                                bb_b[i].at[pl.ds(0, bpc)], sem_b[i]),
               pltpu.async_copy(s_hbm.at[pl.ds(base, bpc)], s_b[i], sem_b[i]))
        return cps

    def chunk_finish(c, cps):
        i = c % N_BUF
        for cp in cps:
            cp.wait()
        wu_v, b_v, s_v = wu_b[i], bb_b[i], s_b[i]
        lanes = lax.iota(jnp.int32, L)

        def group(g, carry):
            uw_p = jnp.zeros((L,), jnp.float32)
            ww_p = jnp.zeros((L,), jnp.float32)
            sw_p = jnp.zeros((L,), jnp.float32)
            r0 = g * L
            for k in range(L):
                r = r0 + k
                uw = jnp.zeros((L,), jnp.float32)
                ww = jnp.zeros((L,), jnp.float32)
                sw = jnp.zeros((L,), jnp.float32)
                for j in range(nj):
                    wj = wu_v[r, pl.ds(j * L, L)]
                    uw = uw + wu_v[r, pl.ds(sdim + j * L, L)] * wj
                    ww = ww + wj * wj
                    sw = sw + s_v[r, pl.ds(j * L, L)] * wj
                mk = lanes == k
                uw_p = jnp.where(mk, jnp.full((L,), jnp.sum(uw)), uw_p)
                ww_p = jnp.where(mk, jnp.full((L,), jnp.sum(ww)), ww_p)
                sw_p = jnp.where(mk, jnp.full((L,), jnp.sum(sw)), sw_p)
            x_p = sw_p + b_v[pl.ds(r0, L)]

            z = jnp.exp(jnp.minimum(uw_p, -1.0))
            y0 = z * (1.0 - z * (0.5 - z * (1.0 / 3.0 - z * 0.25)))
            y1 = y0 - 1.0 + (1.0 + z) * jnp.exp(-y0)
            scale_p = jnp.where(uw_p < -1.0, (y1 - 1.0 - uw_p) / ww_p, 0.0)

            e = jnp.exp(-2.0 * jnp.abs(x_p))
            t0 = (1.0 - e) / (1.0 + e)
            t_p = jnp.where(x_p < 0.0, -t0, t0)

            for k in range(L):
                r = r0 + k
                mk = lanes == k
                scale = jnp.full((L,), jnp.sum(jnp.where(mk, scale_p, 0.0)))
                t = jnp.full((L,), jnp.sum(jnp.where(mk, t_p, 0.0)))
                for j in range(nj):
                    sl = pl.ds(j * L, L)
                    s_v[r, sl] = s_v[r, sl] + (wu_v[r, pl.ds(sdim + j * L, L)]
                                               + scale * wu_v[r, sl]) * t
            return carry

        lax.fori_loop(0, bpc // L, group, 0)

        out_cp[i] = pltpu.async_copy(
            s_v, out_hbm.at[pl.ds(base0 + c * bpc, bpc)], osem_b[i])

    depth = N_BUF - 1
    pend = [chunk_start(c) for c in range(min(depth, N_CHUNK))]
    for c in range(N_CHUNK):
        if c + depth < N_CHUNK:
            pend.append(chunk_start(c + depth))
        chunk_finish(c, pend[c])
    for i in range(N_BUF):
        if out_cp[i] is not None:
            out_cp[i].wait()


def kernel(m, s, w, b, u):
    batch, sdim = s.shape
    n_rows = w.shape[0]
    bpc = batch // (NW * N_CHUNK)
    wu_tab = jnp.concatenate([w, u], axis=1)
    b_flat = b.reshape(n_rows)

    scratch = (
        [pltpu.VMEM((bpc,), jnp.int32) for _ in range(N_BUF)] +
        [pltpu.VMEM((bpc, 2 * sdim), jnp.float32) for _ in range(N_BUF)] +
        [pltpu.VMEM((bpc + L,), jnp.float32) for _ in range(N_BUF)] +
        [pltpu.VMEM((bpc, sdim), jnp.float32) for _ in range(N_BUF)] +
        [pltpu.SemaphoreType.DMA for _ in range(N_BUF)] +
        [pltpu.SemaphoreType.DMA for _ in range(N_BUF)]
    )
    run = pl.kernel(
        _planar_body,
        out_type=jax.ShapeDtypeStruct((batch, sdim), jnp.float32),
        mesh=plsc.VectorSubcoreMesh(core_axis_name="c", subcore_axis_name="s"),
        compiler_params=pltpu.CompilerParams(
            needs_layout_passes=False, use_tc_tiling_on_sc=True),
        scratch_types=scratch,
    )
    return run(m, s, wu_tab, b_flat)
